# baseline (speedup 1.0000x reference)
import functools

import jax
import jax.numpy as jnp
from jax import lax
from jax.experimental import pallas as pl
from jax.experimental.pallas import tpu as pltpu
from jax.experimental.pallas import tpu_sc as plsc

_N = 10000
_E = 320000
_H = 128
_C = 40
_NC = 2
_NS = 16
_NW = _NC * _NS
_B = 128
_CHUNKS = 80
_EPAD = _NW * _CHUNKS * _B
_NACC = _N + 112
_RPS = _NACC // _NS
_NSLOT = 8

_mesh = plsc.VectorSubcoreMesh(core_axis_name="c", subcore_axis_name="s")


@functools.partial(
    pl.kernel,
    out_type=(
        jax.ShapeDtypeStruct((_NACC, _H), jnp.float32),
        jax.ShapeDtypeStruct((_NACC, _H), jnp.float32),
    ),
    mesh=_mesh,
    scratch_types=[
        pltpu.VMEM((_CHUNKS, _B), jnp.int32),
        pltpu.VMEM((_NSLOT, _B), jnp.int32),
        pltpu.VMEM((_NSLOT, _B), jnp.int32),
        pltpu.VMEM((_B, _H), jnp.float32),
        pltpu.VMEM((_B, _H), jnp.float32),
        pltpu.VMEM_SHARED((_NACC, _H), jnp.float32),
        pltpu.SemaphoreType.DMA,
        pltpu.SemaphoreType.DMA,
        pltpu.SemaphoreType.DMA,
        pltpu.SemaphoreType.DMA,
    ],
)
def _spmm(x_hbm, edge_hbm, p0, p1,
          pk_v, src_r, dst_r, gb0, gb1, acc, sem0, sem1, ssem0, ssem1):
    c = lax.axis_index("c")
    s = lax.axis_index("s")
    wid = c * _NS + s
    pltpu.sync_copy(edge_hbm.at[wid], pk_v)
    row0 = s * _RPS

    def unpack(ch):
        ch = jnp.minimum(ch, _CHUNKS - 1)
        slot = jnp.bitwise_and(ch, _NSLOT - 1)
        for jj in range(_B // 16):
            sl = pl.ds(jj * 16, 16)
            pk = pk_v[ch, sl]
            src_r[slot, sl] = jnp.bitwise_and(pk, jnp.int32(0xFFFF))
            dst_r[slot, sl] = jnp.right_shift(pk, jnp.int32(16))

    def slot_of(ch):
        return jnp.bitwise_and(ch, _NSLOT - 1)

    unpack(jnp.int32(0))
    unpack(jnp.int32(1))
    unpack(jnp.int32(2))

    zv = jnp.zeros((16,), jnp.float32)

    def zrow(i, carry):
        for jj in range(_H // 16):
            gb0[i, pl.ds(jj * 16, 16)] = zv
        return carry

    lax.fori_loop(0, _B, zrow, 0)
    for k in range(_RPS // _B):
        pltpu.sync_copy(gb0, acc.at[pl.ds(row0 + k * _B, _B)])
    rem = _RPS % _B
    if rem:
        pltpu.sync_copy(gb0.at[pl.ds(0, rem)],
                        acc.at[pl.ds(row0 + (_RPS // _B) * _B, rem)])
    plsc.subcore_barrier()

    unpack(jnp.int32(3))
    pltpu.async_copy(x_hbm.at[src_r.at[0]], gb0, sem0)
    pltpu.async_copy(x_hbm.at[src_r.at[1]], gb1, sem1)

    def wait_bytes(sem, ref):
        pltpu.make_async_copy(x_hbm.at[pl.ds(0, _B)], ref, sem).wait()

    def body(i, carry):
        j = 2 * i
        wait_bytes(sem0, gb0)
        pltpu.async_copy(gb0, acc.at[dst_r.at[slot_of(j)]], ssem0, add=True)
        unpack(j + 4)
        wait_bytes(sem1, gb1)
        pltpu.async_copy(gb1, acc.at[dst_r.at[slot_of(j + 1)]], ssem1, add=True)
        unpack(j + 5)
        wait_bytes(ssem0, gb0)
        pltpu.async_copy(x_hbm.at[src_r.at[slot_of(j + 2)]], gb0, sem0)
        wait_bytes(ssem1, gb1)
        pltpu.async_copy(x_hbm.at[src_r.at[slot_of(j + 3)]], gb1, sem1)
        return carry

    lax.fori_loop(0, (_CHUNKS - 2) // 2, body, 0)
    jl = _CHUNKS - 1
    wait_bytes(sem0, gb0)
    pltpu.async_copy(gb0, acc.at[dst_r.at[slot_of(jnp.int32(jl - 1))]], ssem0,
                     add=True)
    wait_bytes(sem1, gb1)
    pltpu.async_copy(gb1, acc.at[dst_r.at[slot_of(jnp.int32(jl))]], ssem1,
                     add=True)
    wait_bytes(ssem0, gb0)
    wait_bytes(ssem1, gb1)
    plsc.subcore_barrier()

    @pl.when(c == 0)
    def _():
        pltpu.sync_copy(acc.at[pl.ds(row0, _RPS)], p0.at[pl.ds(row0, _RPS)])

    @pl.when(c == 1)
    def _():
        pltpu.sync_copy(acc.at[pl.ds(row0, _RPS)], p1.at[pl.ds(row0, _RPS)])


_BLK = 1000


def _tc_layer_body(p0_ref, p1_ref, w_ref, b_ref, o_ref):
    z = p0_ref[...] + p1_ref[...]
    y = jnp.dot(z, w_ref[...], preferred_element_type=jnp.float32) + b_ref[...]
    o_ref[...] = jnp.maximum(y, 0.0)


def _tc_out_body(p0_ref, p1_ref, w_ref, b_ref, o_ref):
    z = p0_ref[...] + p1_ref[...]
    x = jnp.dot(z, w_ref[...], preferred_element_type=jnp.float32) + b_ref[...]
    col = lax.broadcasted_iota(jnp.int32, x.shape, 1)
    valid = col < _C
    xm = jnp.where(valid, x, jnp.float32(-3.0e38))
    m = jnp.max(xm, axis=1, keepdims=True)
    e = jnp.where(valid, jnp.exp(x - m), 0.0)
    ssum = jnp.sum(e, axis=1, keepdims=True)
    o_ref[...] = x - m - jnp.log(ssum)


def _tc_call(body, p0, p1, W, b):
    return pl.pallas_call(
        body,
        grid=(_N // _BLK,),
        in_specs=[
            pl.BlockSpec((_BLK, _H), lambda i: (i, 0)),
            pl.BlockSpec((_BLK, _H), lambda i: (i, 0)),
            pl.BlockSpec((_H, _H), lambda i: (0, 0)),
            pl.BlockSpec((1, _H), lambda i: (0, 0)),
        ],
        out_specs=pl.BlockSpec((_BLK, _H), lambda i: (i, 0)),
        out_shape=jax.ShapeDtypeStruct((_N, _H), jnp.float32),
    )(p0, p1, W, b.reshape(1, _H))


def kernel(fea, edge_index, W_in, b_in, W1, b1, W2, b2, W_out, b_out):
    src = edge_index[0]
    dst = edge_index[1]
    pad = _EPAD - _E
    packed = jnp.bitwise_or(src, jnp.left_shift(dst, 16))
    pkp = jnp.pad(packed, (0, pad), constant_values=_N << 16)
    pkp = pkp.reshape(_NW, _CHUNKS, _B)

    p0, p1 = _spmm(fea, pkp)
    x = _tc_call(_tc_layer_body, p0, p1, W_in, b_in)
    p0, p1 = _spmm(x, pkp)
    x = _tc_call(_tc_layer_body, p0, p1, W1, b1)
    p0, p1 = _spmm(x, pkp)
    x = _tc_call(_tc_layer_body, p0, p1, W2, b2)
    p0, p1 = _spmm(x, pkp)

    W_pad = jnp.zeros((_H, _H), W_out.dtype).at[:, :_C].set(W_out)
    b_pad = jnp.zeros((_H,), b_out.dtype).at[:_C].set(b_out)
    out = _tc_call(_tc_out_body, p0, p1, W_pad, b_pad)
    return out[:, :_C]


# speedup vs baseline: 3.0906x; 3.0906x over previous
import functools

import jax
import jax.numpy as jnp
from jax import lax
from jax.experimental import pallas as pl
from jax.experimental.pallas import tpu as pltpu
from jax.experimental.pallas import tpu_sc as plsc

_N = 10000
_E = 320000
_H = 128
_C = 40
_NC = 2
_NS = 16
_NW = _NC * _NS
_B = 128
_CHUNKS = 80
_EPAD = _NW * _CHUNKS * _B
_NACC = _N + 112
_RPS = _NACC // _NS
_NSLOT = 8

_mesh = plsc.VectorSubcoreMesh(core_axis_name="c", subcore_axis_name="s")


@functools.partial(
    pl.kernel,
    out_type=(
        jax.ShapeDtypeStruct((_NACC, _H), jnp.float32),
        jax.ShapeDtypeStruct((_NACC, _H), jnp.float32),
    ),
    mesh=_mesh,
    scratch_types=[
        pltpu.VMEM((_CHUNKS, _B), jnp.int32),
        pltpu.VMEM((_NSLOT, _B), jnp.int32),
        pltpu.VMEM((_NSLOT, _B), jnp.int32),
        pltpu.VMEM((_B, _H), jnp.float32),
        pltpu.VMEM((_B, _H), jnp.float32),
        pltpu.VMEM_SHARED((_NACC, _H), jnp.float32),
        pltpu.SemaphoreType.DMA,
        pltpu.SemaphoreType.DMA,
        pltpu.SemaphoreType.DMA,
        pltpu.SemaphoreType.DMA,
    ],
)
def _spmm(x_hbm, edge_hbm, p0, p1,
          pk_v, src_r, dst_r, gb0, gb1, acc, sem0, sem1, ssem0, ssem1):
    c = lax.axis_index("c")
    s = lax.axis_index("s")
    wid = c * _NS + s
    pltpu.sync_copy(edge_hbm.at[wid], pk_v)
    row0 = s * _RPS

    def unpack(ch):
        ch = jnp.minimum(ch, _CHUNKS - 1)
        slot = jnp.bitwise_and(ch, _NSLOT - 1)
        for jj in range(_B // 16):
            sl = pl.ds(jj * 16, 16)
            pk = pk_v[ch, sl]
            src_r[slot, sl] = jnp.bitwise_and(pk, jnp.int32(0xFFFF))
            dst_r[slot, sl] = jnp.right_shift(pk, jnp.int32(16))

    def slot_of(ch):
        return jnp.bitwise_and(ch, _NSLOT - 1)

    unpack(jnp.int32(0))
    unpack(jnp.int32(1))
    unpack(jnp.int32(2))

    zv = jnp.zeros((16,), jnp.float32)

    def zrow(i, carry):
        for jj in range(_H // 16):
            gb0[i, pl.ds(jj * 16, 16)] = zv
        return carry

    lax.fori_loop(0, _B, zrow, 0)
    for k in range(_RPS // _B):
        pltpu.sync_copy(gb0, acc.at[pl.ds(row0 + k * _B, _B)])
    rem = _RPS % _B
    if rem:
        pltpu.sync_copy(gb0.at[pl.ds(0, rem)],
                        acc.at[pl.ds(row0 + (_RPS // _B) * _B, rem)])
    plsc.subcore_barrier()

    unpack(jnp.int32(3))
    pltpu.async_copy(x_hbm.at[src_r.at[0]], gb0, sem0)
    pltpu.async_copy(x_hbm.at[src_r.at[1]], gb1, sem1)

    def wait_bytes(sem, ref):
        pltpu.make_async_copy(x_hbm.at[pl.ds(0, _B)], ref, sem).wait()

    def body(i, carry):
        j = 2 * i
        wait_bytes(sem0, gb0)
        pltpu.async_copy(gb0, acc.at[dst_r.at[slot_of(j)]], ssem0, add=True)
        unpack(j + 4)
        wait_bytes(sem1, gb1)
        pltpu.async_copy(gb1, acc.at[dst_r.at[slot_of(j + 1)]], ssem1, add=True)
        unpack(j + 5)
        wait_bytes(ssem0, gb0)
        pltpu.async_copy(x_hbm.at[src_r.at[slot_of(j + 2)]], gb0, sem0)
        wait_bytes(ssem1, gb1)
        pltpu.async_copy(x_hbm.at[src_r.at[slot_of(j + 3)]], gb1, sem1)
        return carry

    lax.fori_loop(0, (_CHUNKS - 2) // 2, body, 0)
    jl = _CHUNKS - 1
    wait_bytes(sem0, gb0)
    pltpu.async_copy(gb0, acc.at[dst_r.at[slot_of(jnp.int32(jl - 1))]], ssem0,
                     add=True)
    wait_bytes(sem1, gb1)
    pltpu.async_copy(gb1, acc.at[dst_r.at[slot_of(jnp.int32(jl))]], ssem1,
                     add=True)
    wait_bytes(ssem0, gb0)
    wait_bytes(ssem1, gb1)
    plsc.subcore_barrier()

    @pl.when(c == 0)
    def _():
        pltpu.sync_copy(acc.at[pl.ds(row0, _RPS)], p0.at[pl.ds(row0, _RPS)])

    @pl.when(c == 1)
    def _():
        pltpu.sync_copy(acc.at[pl.ds(row0, _RPS)], p1.at[pl.ds(row0, _RPS)])


_BLK = 1000


def _tc_layer_body(p0_ref, p1_ref, w_ref, b_ref, o_ref):
    z = p0_ref[...] + p1_ref[...]
    y = jnp.dot(z, w_ref[...], preferred_element_type=jnp.float32) + b_ref[...]
    o_ref[...] = jnp.maximum(y, 0.0)


def _tc_out_body(p0_ref, p1_ref, w_ref, b_ref, o_ref):
    z = p0_ref[...] + p1_ref[...]
    x = jnp.dot(z, w_ref[...], preferred_element_type=jnp.float32) + b_ref[...]
    col = lax.broadcasted_iota(jnp.int32, x.shape, 1)
    valid = col < _C
    xm = jnp.where(valid, x, jnp.float32(-3.0e38))
    m = jnp.max(xm, axis=1, keepdims=True)
    e = jnp.where(valid, jnp.exp(x - m), 0.0)
    ssum = jnp.sum(e, axis=1, keepdims=True)
    o_ref[...] = x - m - jnp.log(ssum)


def _tc_call(body, p0, p1, W, b):
    return pl.pallas_call(
        body,
        grid=(_N // _BLK,),
        in_specs=[
            pl.BlockSpec((_BLK, _H), lambda i: (i, 0)),
            pl.BlockSpec((_BLK, _H), lambda i: (i, 0)),
            pl.BlockSpec((_H, _H), lambda i: (0, 0)),
            pl.BlockSpec((1, _H), lambda i: (0, 0)),
        ],
        out_specs=pl.BlockSpec((_BLK, _H), lambda i: (i, 0)),
        out_shape=jax.ShapeDtypeStruct((_N, _H), jnp.float32),
    )(p0, p1, W, b.reshape(1, _H))


def kernel(fea, edge_index, W_in, b_in, W1, b1, W2, b2, W_out, b_out):
    src = edge_index[0]
    dst = edge_index[1]
    pad = _EPAD - _E
    packed = jnp.bitwise_or(src, jnp.left_shift(dst, 16))
    ar = jnp.arange(pad, dtype=jnp.int32)
    pad_src = (ar * 131) % _N
    pad_dst = _N + (ar % (_NACC - _N))
    pad_pk = jnp.bitwise_or(pad_src, jnp.left_shift(pad_dst, 16))
    pkp = jnp.concatenate([packed, pad_pk]).reshape(_NW, _CHUNKS, _B)

    p0, p1 = _spmm(fea, pkp)
    x = _tc_call(_tc_layer_body, p0, p1, W_in, b_in)
    p0, p1 = _spmm(x, pkp)
    x = _tc_call(_tc_layer_body, p0, p1, W1, b1)
    p0, p1 = _spmm(x, pkp)
    x = _tc_call(_tc_layer_body, p0, p1, W2, b2)
    p0, p1 = _spmm(x, pkp)

    W_pad = jnp.zeros((_H, _H), W_out.dtype).at[:, :_C].set(W_out)
    b_pad = jnp.zeros((_H,), b_out.dtype).at[:_C].set(b_out)
    out = _tc_call(_tc_out_body, p0, p1, W_pad, b_pad)
    return out[:, :_C]


# speedup vs baseline: 3.5558x; 1.1505x over previous
import functools

import jax
import jax.numpy as jnp
from jax import lax
from jax.experimental import pallas as pl
from jax.experimental.pallas import tpu as pltpu
from jax.experimental.pallas import tpu_sc as plsc

_N = 10000
_E = 320000
_H = 128
_C = 40
_NC = 2
_NS = 16
_NW = _NC * _NS
_B = 80
_CHUNKS = 126
_EPAD = _NW * _CHUNKS * _B
_NACC = _N + 112
_RPS = _NACC // _NS
_NSLOT = 8

_mesh = plsc.VectorSubcoreMesh(core_axis_name="c", subcore_axis_name="s")


@functools.partial(
    pl.kernel,
    out_type=(
        jax.ShapeDtypeStruct((_NACC, _H), jnp.float32),
        jax.ShapeDtypeStruct((_NACC, _H), jnp.float32),
    ),
    mesh=_mesh,
    scratch_types=[
        pltpu.VMEM((_CHUNKS, _B), jnp.int32),
        pltpu.VMEM((_NSLOT, _B), jnp.int32),
        pltpu.VMEM((_NSLOT, _B), jnp.int32),
        pltpu.VMEM((_B, _H), jnp.float32),
        pltpu.VMEM((_B, _H), jnp.float32),
        pltpu.VMEM((_B, _H), jnp.float32),
        pltpu.VMEM_SHARED((_NACC, _H), jnp.float32),
        pltpu.SemaphoreType.DMA,
        pltpu.SemaphoreType.DMA,
        pltpu.SemaphoreType.DMA,
        pltpu.SemaphoreType.DMA,
        pltpu.SemaphoreType.DMA,
        pltpu.SemaphoreType.DMA,
    ],
)
def _spmm(x_hbm, edge_hbm, p0, p1,
          pk_v, src_r, dst_r, gb0, gb1, gb2, acc,
          sem0, sem1, sem2, ssem0, ssem1, ssem2):
    c = lax.axis_index("c")
    s = lax.axis_index("s")
    wid = c * _NS + s
    pltpu.sync_copy(edge_hbm.at[wid], pk_v)
    row0 = s * _RPS

    def unpack(ch):
        ch = jnp.minimum(ch, _CHUNKS - 1)
        slot = jnp.bitwise_and(ch, _NSLOT - 1)
        for jj in range(_B // 16):
            rsl = pl.ds(jj * 16, 16)
            pk = pk_v[ch, rsl]
            src_r[slot, rsl] = jnp.bitwise_and(pk, jnp.int32(0xFFFF))
            dst_r[slot, rsl] = jnp.right_shift(pk, jnp.int32(16))

    def slot_of(ch):
        return jnp.bitwise_and(ch, _NSLOT - 1)

    for ci in range(6):
        unpack(jnp.int32(ci))

    zv = jnp.zeros((16,), jnp.float32)

    def zrow(i, carry):
        for jj in range(_H // 16):
            gb0[i, pl.ds(jj * 16, 16)] = zv
        return carry

    lax.fori_loop(0, _B, zrow, 0)
    for k in range(_RPS // _B):
        pltpu.sync_copy(gb0, acc.at[pl.ds(row0 + k * _B, _B)])
    rem = _RPS % _B
    if rem:
        pltpu.sync_copy(gb0.at[pl.ds(0, rem)],
                        acc.at[pl.ds(row0 + (_RPS // _B) * _B, rem)])
    plsc.subcore_barrier()

    gbs = (gb0, gb1, gb2)
    sems = (sem0, sem1, sem2)
    ssems = (ssem0, ssem1, ssem2)

    def wait_bytes(sem, ref):
        pltpu.make_async_copy(x_hbm.at[pl.ds(0, _B)], ref, sem).wait()

    for k in range(3):
        pltpu.async_copy(x_hbm.at[src_r.at[k]], gbs[k], sems[k])

    def body(i, carry):
        j = 3 * i
        for k in range(3):
            wait_bytes(sems[k], gbs[k])
            pltpu.async_copy(gbs[k], acc.at[dst_r.at[slot_of(j + k)]],
                             ssems[k], add=True)
        for k in range(3):
            wait_bytes(ssems[k], gbs[k])
            pltpu.async_copy(x_hbm.at[src_r.at[slot_of(j + 3 + k)]],
                             gbs[k], sems[k])
        for k in range(3):
            unpack(j + 6 + k)
        return carry

    lax.fori_loop(0, _CHUNKS // 3 - 1, body, 0)
    jl = _CHUNKS - 3
    for k in range(3):
        wait_bytes(sems[k], gbs[k])
        pltpu.async_copy(gbs[k], acc.at[dst_r.at[slot_of(jnp.int32(jl + k))]],
                         ssems[k], add=True)
    for k in range(3):
        wait_bytes(ssems[k], gbs[k])
    plsc.subcore_barrier()

    @pl.when(c == 0)
    def _():
        pltpu.sync_copy(acc.at[pl.ds(row0, _RPS)], p0.at[pl.ds(row0, _RPS)])

    @pl.when(c == 1)
    def _():
        pltpu.sync_copy(acc.at[pl.ds(row0, _RPS)], p1.at[pl.ds(row0, _RPS)])


_BLK = 1000


def _tc_layer_body(p0_ref, p1_ref, w_ref, b_ref, o_ref):
    z = p0_ref[...] + p1_ref[...]
    y = jnp.dot(z, w_ref[...], preferred_element_type=jnp.float32) + b_ref[...]
    o_ref[...] = jnp.maximum(y, 0.0)


def _tc_out_body(p0_ref, p1_ref, w_ref, b_ref, o_ref):
    z = p0_ref[...] + p1_ref[...]
    x = jnp.dot(z, w_ref[...], preferred_element_type=jnp.float32) + b_ref[...]
    col = lax.broadcasted_iota(jnp.int32, x.shape, 1)
    valid = col < _C
    xm = jnp.where(valid, x, jnp.float32(-3.0e38))
    m = jnp.max(xm, axis=1, keepdims=True)
    e = jnp.where(valid, jnp.exp(x - m), 0.0)
    ssum = jnp.sum(e, axis=1, keepdims=True)
    o_ref[...] = x - m - jnp.log(ssum)


def _tc_call(body, p0, p1, W, b):
    return pl.pallas_call(
        body,
        grid=(_N // _BLK,),
        in_specs=[
            pl.BlockSpec((_BLK, _H), lambda i: (i, 0)),
            pl.BlockSpec((_BLK, _H), lambda i: (i, 0)),
            pl.BlockSpec((_H, _H), lambda i: (0, 0)),
            pl.BlockSpec((1, _H), lambda i: (0, 0)),
        ],
        out_specs=pl.BlockSpec((_BLK, _H), lambda i: (i, 0)),
        out_shape=jax.ShapeDtypeStruct((_N, _H), jnp.float32),
    )(p0, p1, W, b.reshape(1, _H))


def kernel(fea, edge_index, W_in, b_in, W1, b1, W2, b2, W_out, b_out):
    src = edge_index[0]
    dst = edge_index[1]
    pad = _EPAD - _E
    packed = jnp.bitwise_or(src, jnp.left_shift(dst, 16))
    ar = jnp.arange(pad, dtype=jnp.int32)
    pad_src = (ar * 131) % _N
    pad_dst = _N + (ar % (_NACC - _N))
    pad_pk = jnp.bitwise_or(pad_src, jnp.left_shift(pad_dst, 16))
    pkp = jnp.concatenate([packed, pad_pk]).reshape(_NW, _CHUNKS, _B)

    p0, p1 = _spmm(fea, pkp)
    x = _tc_call(_tc_layer_body, p0, p1, W_in, b_in)
    p0, p1 = _spmm(x, pkp)
    x = _tc_call(_tc_layer_body, p0, p1, W1, b1)
    p0, p1 = _spmm(x, pkp)
    x = _tc_call(_tc_layer_body, p0, p1, W2, b2)
    p0, p1 = _spmm(x, pkp)

    W_pad = jnp.zeros((_H, _H), W_out.dtype).at[:, :_C].set(W_out)
    b_pad = jnp.zeros((_H,), b_out.dtype).at[:_C].set(b_out)
    out = _tc_call(_tc_out_body, p0, p1, W_pad, b_pad)
    return out[:, :_C]


# speedup vs baseline: 3.7796x; 1.0630x over previous
import functools

import jax
import jax.numpy as jnp
from jax import lax
from jax.experimental import pallas as pl
from jax.experimental.pallas import tpu as pltpu
from jax.experimental.pallas import tpu_sc as plsc

_N = 10000
_E = 320000
_H = 128
_C = 40
_NC = 2
_NS = 16
_NW = _NC * _NS
_B = 64
_CHUNKS = 160
_EPAD = _NW * _CHUNKS * _B
_NACC = _N + 112
_RPS = _NACC // _NS
_NSLOT = 8

_mesh = plsc.VectorSubcoreMesh(core_axis_name="c", subcore_axis_name="s")


@functools.partial(
    pl.kernel,
    out_type=(
        jax.ShapeDtypeStruct((_NACC, _H), jnp.float32),
        jax.ShapeDtypeStruct((_NACC, _H), jnp.float32),
    ),
    mesh=_mesh,
    scratch_types=[
        pltpu.VMEM((_CHUNKS // 2, 2 * _B), jnp.int32),
        pltpu.VMEM((_NSLOT, _B), jnp.int32),
        pltpu.VMEM((_NSLOT, _B), jnp.int32),
        pltpu.VMEM((_B, _H), jnp.float32),
        pltpu.VMEM((_B, _H), jnp.float32),
        pltpu.VMEM((_B, _H), jnp.float32),
        pltpu.VMEM((_B, _H), jnp.float32),
        pltpu.VMEM_SHARED((_NACC, _H), jnp.float32),
        pltpu.SemaphoreType.DMA,
        pltpu.SemaphoreType.DMA,
        pltpu.SemaphoreType.DMA,
        pltpu.SemaphoreType.DMA,
        pltpu.SemaphoreType.DMA,
        pltpu.SemaphoreType.DMA,
        pltpu.SemaphoreType.DMA,
        pltpu.SemaphoreType.DMA,
    ],
)
def _spmm(x_hbm, edge_hbm, p0, p1,
          pk_v, src_r, dst_r, gb0, gb1, gb2, gb3, acc,
          sem0, sem1, sem2, sem3, ssem0, ssem1, ssem2, ssem3):
    c = lax.axis_index("c")
    s = lax.axis_index("s")
    wid = c * _NS + s
    pltpu.sync_copy(edge_hbm.at[wid], pk_v)
    row0 = s * _RPS

    def unpack(ch, half):
        ch = jnp.minimum(ch, _CHUNKS - 2 + half)
        slot = jnp.bitwise_and(ch, _NSLOT - 1)
        row = jnp.right_shift(ch, 1)
        for jj in range(_B // 16):
            rsl = pl.ds(jj * 16, 16)
            pk = pk_v[row, pl.ds(half * _B + jj * 16, 16)]
            src_r[slot, rsl] = jnp.bitwise_and(pk, jnp.int32(0xFFFF))
            dst_r[slot, rsl] = jnp.right_shift(pk, jnp.int32(16))

    def slot_of(ch):
        return jnp.bitwise_and(ch, _NSLOT - 1)

    for ci in range(8):
        unpack(jnp.int32(ci), ci % 2)

    zv = jnp.zeros((16,), jnp.float32)

    def zrow(i, carry):
        for jj in range(_H // 16):
            gb0[i, pl.ds(jj * 16, 16)] = zv
        return carry

    lax.fori_loop(0, _B, zrow, 0)
    for k in range(_RPS // _B):
        pltpu.sync_copy(gb0, acc.at[pl.ds(row0 + k * _B, _B)])
    rem = _RPS % _B
    if rem:
        pltpu.sync_copy(gb0.at[pl.ds(0, rem)],
                        acc.at[pl.ds(row0 + (_RPS // _B) * _B, rem)])
    plsc.subcore_barrier()

    gbs = (gb0, gb1, gb2, gb3)
    sems = (sem0, sem1, sem2, sem3)
    ssems = (ssem0, ssem1, ssem2, ssem3)

    def wait_bytes(sem, ref):
        pltpu.make_async_copy(x_hbm.at[pl.ds(0, _B)], ref, sem).wait()

    for k in range(4):
        pltpu.async_copy(x_hbm.at[src_r.at[k]], gbs[k], sems[k])

    def body(i, carry):
        j = 4 * i
        for k in range(4):
            wait_bytes(sems[k], gbs[k])
            pltpu.async_copy(gbs[k], acc.at[dst_r.at[slot_of(j + k)]],
                             ssems[k], add=True)
        for k in range(4):
            wait_bytes(ssems[k], gbs[k])
            pltpu.async_copy(x_hbm.at[src_r.at[slot_of(j + 4 + k)]],
                             gbs[k], sems[k])
        for k in range(4):
            unpack(j + 8 + k, k % 2)
        return carry

    lax.fori_loop(0, _CHUNKS // 4 - 1, body, 0)
    jl = _CHUNKS - 4
    for k in range(4):
        wait_bytes(sems[k], gbs[k])
        pltpu.async_copy(gbs[k], acc.at[dst_r.at[slot_of(jnp.int32(jl + k))]],
                         ssems[k], add=True)
    for k in range(4):
        wait_bytes(ssems[k], gbs[k])
    plsc.subcore_barrier()

    @pl.when(c == 0)
    def _():
        pltpu.sync_copy(acc.at[pl.ds(row0, _RPS)], p0.at[pl.ds(row0, _RPS)])

    @pl.when(c == 1)
    def _():
        pltpu.sync_copy(acc.at[pl.ds(row0, _RPS)], p1.at[pl.ds(row0, _RPS)])


_BLK = 1000


def _tc_layer_body(p0_ref, p1_ref, w_ref, b_ref, o_ref):
    z = p0_ref[...] + p1_ref[...]
    y = jnp.dot(z, w_ref[...], preferred_element_type=jnp.float32) + b_ref[...]
    o_ref[...] = jnp.maximum(y, 0.0)


def _tc_out_body(p0_ref, p1_ref, w_ref, b_ref, o_ref):
    z = p0_ref[...] + p1_ref[...]
    x = jnp.dot(z, w_ref[...], preferred_element_type=jnp.float32) + b_ref[...]
    col = lax.broadcasted_iota(jnp.int32, x.shape, 1)
    valid = col < _C
    xm = jnp.where(valid, x, jnp.float32(-3.0e38))
    m = jnp.max(xm, axis=1, keepdims=True)
    e = jnp.where(valid, jnp.exp(x - m), 0.0)
    ssum = jnp.sum(e, axis=1, keepdims=True)
    o_ref[...] = x - m - jnp.log(ssum)


def _tc_call(body, p0, p1, W, b):
    return pl.pallas_call(
        body,
        grid=(_N // _BLK,),
        in_specs=[
            pl.BlockSpec((_BLK, _H), lambda i: (i, 0)),
            pl.BlockSpec((_BLK, _H), lambda i: (i, 0)),
            pl.BlockSpec((_H, _H), lambda i: (0, 0)),
            pl.BlockSpec((1, _H), lambda i: (0, 0)),
        ],
        out_specs=pl.BlockSpec((_BLK, _H), lambda i: (i, 0)),
        out_shape=jax.ShapeDtypeStruct((_N, _H), jnp.float32),
    )(p0, p1, W, b.reshape(1, _H))


def kernel(fea, edge_index, W_in, b_in, W1, b1, W2, b2, W_out, b_out):
    src = edge_index[0]
    dst = edge_index[1]
    pad = _EPAD - _E
    packed = jnp.bitwise_or(src, jnp.left_shift(dst, 16))
    ar = jnp.arange(pad, dtype=jnp.int32)
    pad_src = (ar * 131) % _N
    pad_dst = _N + (ar % (_NACC - _N))
    pad_pk = jnp.bitwise_or(pad_src, jnp.left_shift(pad_dst, 16))
    pkp = jnp.concatenate([packed, pad_pk]).reshape(_NW, _CHUNKS // 2, 2 * _B)

    p0, p1 = _spmm(fea, pkp)
    x = _tc_call(_tc_layer_body, p0, p1, W_in, b_in)
    p0, p1 = _spmm(x, pkp)
    x = _tc_call(_tc_layer_body, p0, p1, W1, b1)
    p0, p1 = _spmm(x, pkp)
    x = _tc_call(_tc_layer_body, p0, p1, W2, b2)
    p0, p1 = _spmm(x, pkp)

    W_pad = jnp.zeros((_H, _H), W_out.dtype).at[:, :_C].set(W_out)
    b_pad = jnp.zeros((_H,), b_out.dtype).at[:_C].set(b_out)
    out = _tc_call(_tc_out_body, p0, p1, W_pad, b_pad)
    return out[:, :_C]
